# baseline (speedup 1.0000x reference)
import functools

import jax
import jax.numpy as jnp
from jax import lax
from jax.experimental import pallas as pl
from jax.experimental.pallas import tpu as pltpu
from jax.experimental.pallas import tpu_sc as plsc

K = 16
_SPLIT_BATCH = False
_SC_NC, _SC_NS = 2, 16
_SC_NW = _SC_NC * _SC_NS


def _sc_gather(table, idx):
    T, D = table.shape
    (M,) = idx.shape
    m_w = M // _SC_NW
    assert m_w * _SC_NW == M
    chunk = min(m_w, 128)
    nchunks = m_w // chunk
    assert nchunks * chunk == m_w
    NB = max(1, min(nchunks, (400 * 1024) // (chunk * D * 4)))

    idx2 = idx.reshape(M // chunk, chunk)
    mesh = plsc.VectorSubcoreMesh(core_axis_name="c", subcore_axis_name="s")

    @functools.partial(
        pl.kernel,
        mesh=mesh,
        out_type=jax.ShapeDtypeStruct((M, D), jnp.float32),
        scratch_types=[pltpu.VMEM((nchunks, chunk), jnp.int32)]
        + [pltpu.VMEM((chunk, D), jnp.float32) for _ in range(NB)]
        + [pltpu.SemaphoreType.DMA for _ in range(2 * NB)],
    )
    def gk(table_hbm, idx_hbm, out_hbm, idx_all, *rest):
        rows = rest[:NB]
        gsems = rest[NB:2 * NB]
        ssems = rest[2 * NB:3 * NB]
        wid = lax.axis_index("s") * _SC_NC + lax.axis_index("c")
        rbase = wid * nchunks
        pltpu.sync_copy(idx_hbm.at[pl.ds(rbase, nchunks)], idx_all)
        copies = [None] * nchunks
        stores = [None] * nchunks
        for p in range(min(NB, nchunks)):
            copies[p] = pltpu.async_copy(
                table_hbm.at[idx_all.at[p]], rows[p], gsems[p])
        for c in range(nchunks):
            s = c % NB
            copies[c].wait()
            stores[c] = pltpu.async_copy(
                rows[s], out_hbm.at[pl.ds((rbase + c) * chunk, chunk)],
                ssems[s])
            nxt = c + NB
            if nxt < nchunks:
                stores[c].wait()
                copies[nxt] = pltpu.async_copy(
                    table_hbm.at[idx_all.at[nxt]], rows[s], gsems[s])
        for c in range(max(0, nchunks - NB), nchunks):
            stores[c].wait()

    return gk(table, idx2)


def _gather_kv(kv, idx_flat):
    B_, N_, D = kv.shape
    out = _sc_gather(kv.reshape(B_ * N_, D), idx_flat)
    return out.reshape(B_, N_ * K, D)


def _knn(aug_a, aug_b, pos16_a, pos16_b, Ra):
    B_, Na, _ = aug_a.shape
    Nb = aug_b.shape[1]

    def body(a_ref, b_ref, pa_ref, pb_ref, idx_ref, pd_ref):
        b_id = pl.program_id(0)
        a = a_ref[0]
        bb = b_ref[0]
        d = lax.dot_general(a, bb, (((1,), (1,)), ((), ())),
                            preferred_element_type=jnp.float32)
        col = lax.broadcasted_iota(jnp.int32, (Ra, Nb), 1)
        dbits = lax.bitcast_convert_type(d, jnp.int32)
        packed = (dbits & jnp.int32(~0x7FF)) | col
        dead = jnp.int32(0x7FFFFFFF)
        pa = pa_ref[0]
        keys = []
        pds = []
        for _ in range(K):
            m = jnp.min(packed, axis=1, keepdims=True)
            keys.append(m)
            eq = packed == m
            sel = eq.astype(jnp.float32)
            psel = jnp.dot(sel, pb_ref[0],
                           preferred_element_type=jnp.float32)
            pds.append((pa - psel)[:, None, :])
            packed = jnp.where(eq, dead, packed)
        kcat = jnp.concatenate(keys, axis=1)
        idx = kcat & jnp.int32(0x7FF)
        idx_ref[0] = idx + b_id * Nb
        pd_ref[0] = jnp.concatenate(pds, axis=1)

    return pl.pallas_call(
        body,
        grid=(B_, Na // Ra),
        in_specs=[
            pl.BlockSpec((1, Ra, 8), lambda b, i: (b, i, 0)),
            pl.BlockSpec((1, Nb, 8), lambda b, i: (b, 0, 0)),
            pl.BlockSpec((1, Ra, 16), lambda b, i: (b, i, 0)),
            pl.BlockSpec((1, Nb, 16), lambda b, i: (b, 0, 0)),
        ],
        out_specs=[
            pl.BlockSpec((1, Ra, K), lambda b, i: (b, i, 0)),
            pl.BlockSpec((1, Ra, K, 16), lambda b, i: (b, i, 0, 0)),
        ],
        out_shape=[
            jax.ShapeDtypeStruct((B_, Na, K), jnp.int32),
            jax.ShapeDtypeStruct((B_, Na, K, 16), jnp.float32),
        ],
    )(aug_a, aug_b, pos16_a, pos16_b)


def _full(shape):
    nd = len(shape)
    return pl.BlockSpec(shape, lambda b, i, _nd=nd: (0,) * _nd)


def _bspec(C):
    return pl.BlockSpec((1, 1, C), lambda b, i: (b, 0, 0))


def _mod_qkv_refs(out, e_refs, o_refs):
    sh2, sc2, wq2, bq2, wkv2, bkv2 = e_refs
    xm2 = out * (1.0 + sc2[0]) + sh2[0]
    o_refs[0][0] = xm2
    o_refs[1][0] = jnp.dot(xm2, wq2[...],
                           preferred_element_type=jnp.float32) + bq2[...]
    o_refs[2][0] = jnp.dot(xm2, wkv2[...],
                           preferred_element_type=jnp.float32) + bkv2[...]


def _nxt_arrays(nxt):
    sh, sc, lq, lkv = nxt
    return [sh, sc, lq['w'], lq['b'][None, :], lkv['w'], lkv['b'][None, :]]


def _nxt_specs(nxt, C):
    sh, sc, lq, lkv = nxt
    return [_bspec(C), _bspec(C), _full(lq['w'].shape),
            _full(lq['b'][None, :].shape), _full(lkv['w'].shape),
            _full(lkv['b'][None, :].shape)]


def _qkv_shapes(B_, N_, C2):
    return [jax.ShapeDtypeStruct((B_, N_, C2), jnp.float32),
            jax.ShapeDtypeStruct((B_, N_, C2), jnp.float32),
            jax.ShapeDtypeStruct((B_, N_, 2 * C2), jnp.float32)]


def _row_specs(shapes, R):
    return [pl.BlockSpec((1, R, s.shape[2]), lambda b, i: (b, i, 0))
            for s in shapes]


def _ln(h, g, b):
    m = jnp.mean(h, axis=-1, keepdims=True)
    v = jnp.mean((h - m) * (h - m), axis=-1, keepdims=True)
    return (h - m) / jnp.sqrt(v + 1e-5) * g + b


def _projin_qkv(aug, win, bin_, nxt, R):
    B_, N_, _ = aug.shape
    C = win.shape[1]

    def body(a_ref, win_ref, bin_ref, sh_ref, sc_ref, wq_ref, bq_ref,
             wkv_ref, bkv_ref, xm_ref, q_ref, kv_ref):
        x = jnp.dot(a_ref[0], win_ref[...],
                    preferred_element_type=jnp.float32) + bin_ref[...]
        _mod_qkv_refs(x, (sh_ref, sc_ref, wq_ref, bq_ref, wkv_ref, bkv_ref),
                      (xm_ref, q_ref, kv_ref))

    oshapes = _qkv_shapes(B_, N_, C)
    return pl.pallas_call(
        body,
        grid=(B_, N_ // R),
        in_specs=[pl.BlockSpec((1, R, 8), lambda b, i: (b, i, 0)),
                  _full(win.shape), _full(bin_.shape)] + _nxt_specs(nxt, C),
        out_specs=_row_specs(oshapes, R),
        out_shape=oshapes,
    )(aug, win, bin_, *_nxt_arrays(nxt))


def _attn_ffn(xm, q, kvn, pd, wts, R, nxt=None, proj=None):
    B_, N_, C = xm.shape
    RK = R * K
    n_w = len(wts)
    if nxt is not None:
        extra = _nxt_arrays(nxt)
        e_specs = _nxt_specs(nxt, C)
        oshapes = _qkv_shapes(B_, N_, C)
    elif proj is not None:
        extra = list(proj)
        e_specs = [_full(proj[0].shape), _full(proj[1].shape)]
        oshapes = [jax.ShapeDtypeStruct((B_, N_, proj[0].shape[1]),
                                        jnp.float32)]
    else:
        extra = []
        e_specs = []
        oshapes = [jax.ShapeDtypeStruct((B_, N_, C), jnp.float32)]

    def body(*refs):
        xm_ref, q_ref, kvn_ref, pd_ref = refs[:4]
        w_refs = refs[4:4 + n_w]
        e_refs = refs[4 + n_w:4 + n_w + len(extra)]
        o_refs = refs[4 + n_w + len(extra):]
        (w1_ref, b1_ref, w2_ref, b2_ref, wa1_ref, ba1_ref, wa2_ref, ba2_ref,
         wo_ref, bo_ref, wf1_ref, bf1_ref, wf2_ref, bf2_ref,
         g1_ref, be1_ref, g2_ref, be2_ref) = w_refs
        kvn_ = kvn_ref[0]
        kn = kvn_[:, :C]
        vn = kvn_[:, C:]
        pe = jnp.dot(jax.nn.relu(
            jnp.dot(pd_ref[0], w1_ref[...],
                    preferred_element_type=jnp.float32)
            + b1_ref[...]), w2_ref[...],
            preferred_element_type=jnp.float32) + b2_ref[...]
        q_ = q_ref[0]
        q_rep = jnp.broadcast_to(q_[:, None, :], (R, K, C)).reshape(RK, C)
        rel = kn - q_rep + pe
        t = jnp.dot(jax.nn.relu(
            jnp.dot(rel, wa1_ref[...], preferred_element_type=jnp.float32)
            + ba1_ref[...]), wa2_ref[...],
            preferred_element_type=jnp.float32) + ba2_ref[...]
        a3 = t.reshape(R, K, C)
        mx = jnp.max(a3, axis=1, keepdims=True)
        e = jnp.exp(a3 - mx)
        p = e / jnp.sum(e, axis=1, keepdims=True)
        vpe = (vn + pe).reshape(R, K, C)
        agg = jnp.sum(p * vpe, axis=1)
        attn = jnp.dot(agg, wo_ref[...],
                       preferred_element_type=jnp.float32) + bo_ref[...]
        x1 = xm_ref[0] + attn
        h = jnp.dot(x1, wf1_ref[...],
                    preferred_element_type=jnp.float32) + bf1_ref[...]
        h = _ln(h, g1_ref[...], be1_ref[...])
        h = jax.nn.gelu(h)
        h = jnp.dot(h, wf2_ref[...],
                    preferred_element_type=jnp.float32) + bf2_ref[...]
        h = _ln(h, g2_ref[...], be2_ref[...])
        out = x1 + h
        if nxt is not None:
            _mod_qkv_refs(out, e_refs, o_refs)
        elif proj is not None:
            o_refs[0][0] = jnp.dot(out, e_refs[0][...],
                                   preferred_element_type=jnp.float32) \
                + e_refs[1][...]
        else:
            o_refs[0][0] = out

    res = pl.pallas_call(
        body,
        grid=(B_, N_ // R),
        in_specs=[
            pl.BlockSpec((1, R, C), lambda b, i: (b, i, 0)),
            pl.BlockSpec((1, R, C), lambda b, i: (b, i, 0)),
            pl.BlockSpec((1, RK, 2 * C), lambda b, i: (b, i, 0)),
            pl.BlockSpec((1, RK, 16), lambda b, i: (b, i, 0)),
        ] + [_full(w.shape) for w in wts] + e_specs,
        out_specs=_row_specs(oshapes, R),
        out_shape=oshapes,
    )(xm, q, kvn, pd, *wts, *extra)
    return res[0] if len(res) == 1 else tuple(res)


def _slot(x, pos16, sp, wd, bd, nxt):
    B_, N_, C = x.shape
    S = sp['slots'].shape[0]
    Cd = wd.shape[1]

    wlist = [sp['slots'],
             sp['wq']['w'], sp['wq']['b'][None, :],
             sp['wk']['w'], sp['wk']['b'][None, :],
             sp['wv']['w'], sp['wv']['b'][None, :],
             sp['mlp1']['w'], sp['mlp1']['b'][None, :],
             sp['mlp2']['w'], sp['mlp2']['b'][None, :],
             wd, bd]

    def body(*refs):
        x_ref, pos_ref = refs[:2]
        (slots_ref, wq_ref, bq_ref, wk_ref, bk_ref, wv_ref, bv_ref,
         m1_ref, bm1_ref, m2_ref, bm2_ref, wd_ref, bd_ref) = refs[2:15]
        e_refs = refs[15:21]
        xm_ref, q2_ref, kv2_ref, posn_ref = refs[21:]
        x_ = x_ref[0]
        qs = jnp.dot(slots_ref[...], wq_ref[...],
                     preferred_element_type=jnp.float32) + bq_ref[...]
        kk = jnp.dot(x_, wk_ref[...],
                     preferred_element_type=jnp.float32) + bk_ref[...]
        vv = jnp.dot(x_, wv_ref[...],
                     preferred_element_type=jnp.float32) + bv_ref[...]
        logits = lax.dot_general(qs, kk, (((1,), (1,)), ((), ())),
                                 preferred_element_type=jnp.float32) / jnp.sqrt(
                                     jnp.float32(C))
        mx = jnp.max(logits, axis=0, keepdims=True)
        e = jnp.exp(logits - mx)
        attn = e / jnp.sum(e, axis=0, keepdims=True)
        w = attn / (jnp.sum(attn, axis=1, keepdims=True) + 1e-8)
        upd = jnp.dot(w, vv, preferred_element_type=jnp.float32)
        t = jnp.dot(jax.nn.relu(
            jnp.dot(upd, m1_ref[...], preferred_element_type=jnp.float32)
            + bm1_ref[...]), m2_ref[...],
            preferred_element_type=jnp.float32) + bm2_ref[...]
        xn = upd + t
        xd = jnp.dot(xn, wd_ref[...],
                     preferred_element_type=jnp.float32) + bd_ref[...]
        _mod_qkv_refs(xd, e_refs, (xm_ref, q2_ref, kv2_ref))
        posn_ref[0] = jnp.dot(w, pos_ref[0],
                              preferred_element_type=jnp.float32)

    oshapes = _qkv_shapes(B_, S, Cd) + [
        jax.ShapeDtypeStruct((B_, S, 16), jnp.float32)]
    return pl.pallas_call(
        body,
        grid=(B_, 1),
        in_specs=[
            pl.BlockSpec((1, N_, C), lambda b, i: (b, 0, 0)),
            pl.BlockSpec((1, N_, 16), lambda b, i: (b, 0, 0)),
        ] + [_full(w.shape) for w in wlist] + _nxt_specs(nxt, Cd),
        out_specs=_row_specs(oshapes, S),
        out_shape=oshapes,
    )(x, pos16, *wlist, *_nxt_arrays(nxt))


def _interp(aug_a, aug_b, x_src, x_skip, wu, bu, nxt, R):
    B_, N_, Cs = x_skip.shape
    S = x_src.shape[1]

    def body(*refs):
        a_ref, b_ref, xsrc_ref, xs_ref, wu_ref, bu_ref = refs[:6]
        e_refs = refs[6:12]
        o_refs = refs[12:]
        d = lax.dot_general(a_ref[0], b_ref[0], (((1,), (1,)), ((), ())),
                            preferred_element_type=jnp.float32)
        col = lax.broadcasted_iota(jnp.int32, (R, S), 1)
        dbits = lax.bitcast_convert_type(d, jnp.int32)
        packed = (dbits & jnp.int32(~0x7FF)) | col
        dead = jnp.int32(0x7FFFFFFF)
        keys = []
        for _ in range(K):
            m = jnp.min(packed, axis=1, keepdims=True)
            keys.append(m)
            packed = jnp.where(packed == m, dead, packed)
        kcat = jnp.concatenate(keys, axis=1)
        d2 = lax.bitcast_convert_type(kcat & jnp.int32(~0x7FF), jnp.float32)
        dist = jnp.sqrt(jnp.maximum(d2, 0.0) + 1e-12)
        w = 1.0 / (dist + 1e-8)
        w = w / jnp.sum(w, axis=1, keepdims=True)
        wm = jnp.zeros((R, S), jnp.float32)
        for r in range(K):
            am = kcat[:, r:r + 1] & jnp.int32(0x7FF)
            wm = jnp.where(col == am, w[:, r:r + 1], wm)
        itp = jnp.dot(wm, xsrc_ref[0],
                      preferred_element_type=jnp.float32)
        out = jnp.dot(itp, wu_ref[...],
                      preferred_element_type=jnp.float32) \
            + bu_ref[...] + xs_ref[0]
        _mod_qkv_refs(out, e_refs, o_refs)

    oshapes = _qkv_shapes(B_, N_, Cs)
    return pl.pallas_call(
        body,
        grid=(B_, N_ // R),
        in_specs=[
            pl.BlockSpec((1, R, 8), lambda b, i: (b, i, 0)),
            pl.BlockSpec((1, S, 8), lambda b, i: (b, 0, 0)),
            pl.BlockSpec((1, S, x_src.shape[2]), lambda b, i: (b, 0, 0)),
            pl.BlockSpec((1, R, Cs), lambda b, i: (b, i, 0)),
            _full(wu.shape), _full(bu.shape),
        ] + _nxt_specs(nxt, Cs),
        out_specs=_row_specs(oshapes, R),
        out_shape=oshapes,
    )(aug_a, aug_b, x_src, x_skip, wu, bu, *_nxt_arrays(nxt))


def _prep_pos(p):
    n2 = jnp.sum(p * p, axis=-1, keepdims=True)
    one = jnp.ones_like(n2)
    z = jnp.zeros_like(n2)
    z3 = jnp.concatenate([z, z, z], -1)
    aug_a = jnp.concatenate([p, n2, one, z3], -1)
    aug_b = jnp.concatenate([-2.0 * p, one, n2, z3], -1)
    pos16 = jnp.concatenate([p, z, z3, z3, z3, z3], -1)
    return aug_a, aug_b, pos16


def _adapt_np(cond, p):
    h = cond @ p['lin1']['w'] + p['lin1']['b']
    m = jnp.mean(h, axis=-1, keepdims=True)
    v = jnp.var(h, axis=-1, keepdims=True)
    h = (h - m) / jnp.sqrt(v + 1e-5) * p['ln']['g'] + p['ln']['b']
    h = h @ p['lin2']['w'] + p['lin2']['b']
    sh, sc = jnp.split(h, 2, axis=1)
    return sh[:, None, :], sc[:, None, :]


def _pad_rows(w, rows):
    return jnp.concatenate(
        [w, jnp.zeros((rows - w.shape[0], w.shape[1]), w.dtype)], axis=0)


def _pad_cols(w, cols):
    return jnp.concatenate(
        [w, jnp.zeros((w.shape[0], cols - w.shape[1]), w.dtype)], axis=1)


def _layer_wts(p):
    return [
        _pad_rows(p['pos1']['w'], 16), p['pos1']['b'][None, :],
        p['pos2']['w'], p['pos2']['b'][None, :],
        p['attn1']['w'], p['attn1']['b'][None, :],
        p['attn2']['w'], p['attn2']['b'][None, :],
        p['fc_out']['w'], p['fc_out']['b'][None, :],
        p['ffn1']['w'], p['ffn1']['b'][None, :],
        p['ffn2']['w'], p['ffn2']['b'][None, :],
        p['ffn_ln1']['g'][None, :], p['ffn_ln1']['b'][None, :],
        p['ffn_ln2']['g'][None, :], p['ffn_ln2']['b'][None, :],
    ]


def _nxt_of(sh_sc, lp):
    sh, sc = sh_sc
    return (sh, sc, lp['fc_q'], lp['fc_kv'])


def kernel(pos, cond, params):
    if _SPLIT_BATCH:
        outs = [_forward_one(pos[b:b + 1], cond[b:b + 1], params)
                for b in range(pos.shape[0])]
        return jnp.concatenate(outs, axis=0)
    return _forward_one(pos, cond, params)


def _forward_one(pos, cond, params):
    B_, N_, _ = pos.shape
    C0, C1 = 64, 128
    S = 512
    R0, R1 = 512, 256

    aug_a0, aug_b0, pos16_0 = _prep_pos(pos)

    sh_sc_down = [_adapt_np(cond, p) for p in params['adapt_down']]
    sh_sc_up = [_adapt_np(cond, p) for p in params['adapt_up']]

    td0, td1 = params['tf_down'][0], params['tf_down'][1]
    tu0 = params['tf_up'][0]

    win = _pad_rows(params['proj_in']['w'], 8)
    xm, q, kv = _projin_qkv(aug_a0, win, params['proj_in']['b'][None, :],
                            _nxt_of(sh_sc_down[0], td0[0]), R0)

    idx0, pd0 = _knn(aug_a0, aug_b0, pos16_0, pos16_0, 512)
    idx0_flat = idx0.reshape(-1)
    pd0 = pd0.reshape(B_, N_ * K, 16)

    kvn = _gather_kv(kv, idx0_flat)
    xm, q, kv = _attn_ffn(xm, q, kvn, pd0, _layer_wts(td0[0]), R0,
                          nxt=_nxt_of(sh_sc_down[1], td0[1]))
    kvn = _gather_kv(kv, idx0_flat)
    x0_skip = _attn_ffn(xm, q, kvn, pd0, _layer_wts(td0[1]), R0)

    xm, q, kv, pos1_16 = _slot(
        x0_skip, pos16_0, params['down'][0],
        params['proj_down'][0]['w'], params['proj_down'][0]['b'][None, :],
        _nxt_of(sh_sc_down[2], td1[0]))
    pos1 = pos1_16[..., :3]
    aug_a1, aug_b1, pos16_1 = _prep_pos(pos1)

    idx1, pd1 = _knn(aug_a1, aug_b1, pos16_1, pos16_1, 512)
    idx1_flat = idx1.reshape(-1)
    pd1 = pd1.reshape(B_, S * K, 16)

    kvn = _gather_kv(kv, idx1_flat)
    xm, q, kv = _attn_ffn(xm, q, kvn, pd1, _layer_wts(td1[0]), R1,
                          nxt=_nxt_of(sh_sc_down[3], td1[1]))
    kvn = _gather_kv(kv, idx1_flat)
    x1_out = _attn_ffn(xm, q, kvn, pd1, _layer_wts(td1[1]), R1)

    xm, q, kv = _interp(aug_a0, aug_b1, x1_out, x0_skip,
                        params['proj_up'][0]['w'],
                        params['proj_up'][0]['b'][None, :],
                        _nxt_of(sh_sc_up[-1], tu0[0]), R0)

    kvn = _gather_kv(kv, idx0_flat)
    xm, q, kv = _attn_ffn(xm, q, kvn, pd0, _layer_wts(tu0[0]), R0,
                          nxt=_nxt_of(sh_sc_up[0], tu0[1]))
    kvn = _gather_kv(kv, idx0_flat)

    wout = _pad_cols(params['proj_out']['w'], 128)
    bout = _pad_cols(params['proj_out']['b'][None, :], 128)
    out = _attn_ffn(xm, q, kvn, pd0, _layer_wts(tu0[1]), R0,
                    proj=(wout, bout))
    return out[..., :1]


# speedup vs baseline: 1.0143x; 1.0143x over previous
import functools

import jax
import jax.numpy as jnp
from jax import lax
from jax.experimental import pallas as pl
from jax.experimental.pallas import tpu as pltpu
from jax.experimental.pallas import tpu_sc as plsc

K = 16
_SC_NC, _SC_NS = 2, 16
_SC_NW = _SC_NC * _SC_NS


def _sc_gather(table, idx):
    T, D = table.shape
    (M,) = idx.shape
    m_w = M // _SC_NW
    assert m_w * _SC_NW == M
    chunk = min(m_w, 128)
    nchunks = m_w // chunk
    assert nchunks * chunk == m_w
    NB = max(1, min(nchunks, (400 * 1024) // (chunk * D * 4)))

    idx2 = idx.reshape(M // chunk, chunk)
    mesh = plsc.VectorSubcoreMesh(core_axis_name="c", subcore_axis_name="s")

    @functools.partial(
        pl.kernel,
        mesh=mesh,
        out_type=jax.ShapeDtypeStruct((M, D), jnp.float32),
        scratch_types=[pltpu.VMEM((nchunks, chunk), jnp.int32)]
        + [pltpu.VMEM((chunk, D), jnp.float32) for _ in range(NB)]
        + [pltpu.SemaphoreType.DMA for _ in range(2 * NB)],
    )
    def gk(table_hbm, idx_hbm, out_hbm, idx_all, *rest):
        rows = rest[:NB]
        gsems = rest[NB:2 * NB]
        ssems = rest[2 * NB:3 * NB]
        wid = lax.axis_index("s") * _SC_NC + lax.axis_index("c")
        rbase = wid * nchunks
        pltpu.sync_copy(idx_hbm.at[pl.ds(rbase, nchunks)], idx_all)
        copies = [None] * nchunks
        stores = [None] * nchunks
        for p in range(min(NB, nchunks)):
            copies[p] = pltpu.async_copy(
                table_hbm.at[idx_all.at[p]], rows[p], gsems[p])
        for c in range(nchunks):
            s = c % NB
            copies[c].wait()
            stores[c] = pltpu.async_copy(
                rows[s], out_hbm.at[pl.ds((rbase + c) * chunk, chunk)],
                ssems[s])
            nxt = c + NB
            if nxt < nchunks:
                stores[c].wait()
                copies[nxt] = pltpu.async_copy(
                    table_hbm.at[idx_all.at[nxt]], rows[s], gsems[s])
        for c in range(max(0, nchunks - NB), nchunks):
            stores[c].wait()

    return gk(table, idx2)


def _gather_kv(kv, idx_flat):
    B_, N_, D = kv.shape
    out = _sc_gather(kv.reshape(B_ * N_, D), idx_flat)
    return out.reshape(B_, N_ * K, D)


def _knn(aug_a, aug_b, pos16_a, pos16_b, Ra):
    B_, Na, _ = aug_a.shape
    Nb = aug_b.shape[1]

    def body(a_ref, b_ref, pa_ref, pb_ref, idx_ref, pd_ref):
        b_id = pl.program_id(0)
        a = a_ref[0]
        bb = b_ref[0]
        d = lax.dot_general(a, bb, (((1,), (1,)), ((), ())),
                            preferred_element_type=jnp.float32)
        col = lax.broadcasted_iota(jnp.int32, (Ra, Nb), 1)
        dbits = lax.bitcast_convert_type(d, jnp.int32)
        packed = (dbits & jnp.int32(~0x7FF)) | col
        dead = jnp.int32(0x7FFFFFFF)
        pa = pa_ref[0]
        keys = []
        pds = []
        for _ in range(K):
            m = jnp.min(packed, axis=1, keepdims=True)
            keys.append(m)
            eq = packed == m
            sel = eq.astype(jnp.float32)
            psel = jnp.dot(sel, pb_ref[0],
                           preferred_element_type=jnp.float32)
            pds.append((pa - psel)[:, None, :])
            packed = jnp.where(eq, dead, packed)
        kcat = jnp.concatenate(keys, axis=1)
        idx = kcat & jnp.int32(0x7FF)
        idx_ref[0] = idx + b_id * Nb
        pd_ref[0] = jnp.concatenate(pds, axis=1)

    return pl.pallas_call(
        body,
        grid=(B_, Na // Ra),
        in_specs=[
            pl.BlockSpec((1, Ra, 8), lambda b, i: (b, i, 0)),
            pl.BlockSpec((1, Nb, 8), lambda b, i: (b, 0, 0)),
            pl.BlockSpec((1, Ra, 16), lambda b, i: (b, i, 0)),
            pl.BlockSpec((1, Nb, 16), lambda b, i: (b, 0, 0)),
        ],
        out_specs=[
            pl.BlockSpec((1, Ra, K), lambda b, i: (b, i, 0)),
            pl.BlockSpec((1, Ra, K, 16), lambda b, i: (b, i, 0, 0)),
        ],
        out_shape=[
            jax.ShapeDtypeStruct((B_, Na, K), jnp.int32),
            jax.ShapeDtypeStruct((B_, Na, K, 16), jnp.float32),
        ],
    )(aug_a, aug_b, pos16_a, pos16_b)


def _full(shape):
    nd = len(shape)
    return pl.BlockSpec(shape, lambda b, i, _nd=nd: (0,) * _nd)


def _bspec(C):
    return pl.BlockSpec((1, 1, C), lambda b, i: (b, 0, 0))


def _mod_qkv_refs(out, e_refs, o_refs):
    sh2, sc2, wq2, bq2, wkv2, bkv2 = e_refs
    xm2 = out * (1.0 + sc2[0]) + sh2[0]
    o_refs[0][0] = xm2
    o_refs[1][0] = jnp.dot(xm2, wq2[...],
                           preferred_element_type=jnp.float32) + bq2[...]
    o_refs[2][0] = jnp.dot(xm2, wkv2[...],
                           preferred_element_type=jnp.float32) + bkv2[...]


def _nxt_arrays(nxt):
    sh, sc, lq, lkv = nxt
    return [sh, sc, lq['w'], lq['b'][None, :], lkv['w'], lkv['b'][None, :]]


def _nxt_specs(nxt, C):
    sh, sc, lq, lkv = nxt
    return [_bspec(C), _bspec(C), _full(lq['w'].shape),
            _full(lq['b'][None, :].shape), _full(lkv['w'].shape),
            _full(lkv['b'][None, :].shape)]


def _qkv_shapes(B_, N_, C2):
    return [jax.ShapeDtypeStruct((B_, N_, C2), jnp.float32),
            jax.ShapeDtypeStruct((B_, N_, C2), jnp.float32),
            jax.ShapeDtypeStruct((B_, N_, 2 * C2), jnp.float32)]


def _row_specs(shapes, R):
    return [pl.BlockSpec((1, R, s.shape[2]), lambda b, i: (b, i, 0))
            for s in shapes]


def _ln(h, g, b):
    m = jnp.mean(h, axis=-1, keepdims=True)
    v = jnp.mean((h - m) * (h - m), axis=-1, keepdims=True)
    return (h - m) / jnp.sqrt(v + 1e-5) * g + b


def _projin_qkv(aug, win, bin_, nxt, R):
    B_, N_, _ = aug.shape
    C = win.shape[1]

    def body(a_ref, win_ref, bin_ref, sh_ref, sc_ref, wq_ref, bq_ref,
             wkv_ref, bkv_ref, xm_ref, q_ref, kv_ref):
        x = jnp.dot(a_ref[0], win_ref[...],
                    preferred_element_type=jnp.float32) + bin_ref[...]
        _mod_qkv_refs(x, (sh_ref, sc_ref, wq_ref, bq_ref, wkv_ref, bkv_ref),
                      (xm_ref, q_ref, kv_ref))

    oshapes = _qkv_shapes(B_, N_, C)
    return pl.pallas_call(
        body,
        grid=(B_, N_ // R),
        in_specs=[pl.BlockSpec((1, R, 8), lambda b, i: (b, i, 0)),
                  _full(win.shape), _full(bin_.shape)] + _nxt_specs(nxt, C),
        out_specs=_row_specs(oshapes, R),
        out_shape=oshapes,
    )(aug, win, bin_, *_nxt_arrays(nxt))


def _attn_ffn(xm, q, kvn, pd, wts, R, nxt=None, proj=None):
    B_, N_, C = xm.shape
    RK = R * K
    n_w = len(wts)
    if nxt is not None:
        extra = _nxt_arrays(nxt)
        e_specs = _nxt_specs(nxt, C)
        oshapes = _qkv_shapes(B_, N_, C)
    elif proj is not None:
        extra = list(proj)
        e_specs = [_full(proj[0].shape), _full(proj[1].shape)]
        oshapes = [jax.ShapeDtypeStruct((B_, N_, proj[0].shape[1]),
                                        jnp.float32)]
    else:
        extra = []
        e_specs = []
        oshapes = [jax.ShapeDtypeStruct((B_, N_, C), jnp.float32)]

    def body(*refs):
        xm_ref, q_ref, kvn_ref, pd_ref = refs[:4]
        w_refs = refs[4:4 + n_w]
        e_refs = refs[4 + n_w:4 + n_w + len(extra)]
        o_refs = refs[4 + n_w + len(extra):]
        (w1_ref, b1_ref, w2_ref, b2_ref, wa1_ref, ba1_ref, wa2_ref, ba2_ref,
         wo_ref, bo_ref, wf1_ref, bf1_ref, wf2_ref, bf2_ref,
         g1_ref, be1_ref, g2_ref, be2_ref) = w_refs
        kvn_ = kvn_ref[0]
        kn = kvn_[:, :C]
        vn = kvn_[:, C:]
        pe = jnp.dot(jax.nn.relu(
            jnp.dot(pd_ref[0], w1_ref[...],
                    preferred_element_type=jnp.float32)
            + b1_ref[...]), w2_ref[...],
            preferred_element_type=jnp.float32) + b2_ref[...]
        q_ = q_ref[0]
        q_rep = jnp.broadcast_to(q_[:, None, :], (R, K, C)).reshape(RK, C)
        rel = kn - q_rep + pe
        t = jnp.dot(jax.nn.relu(
            jnp.dot(rel, wa1_ref[...], preferred_element_type=jnp.float32)
            + ba1_ref[...]), wa2_ref[...],
            preferred_element_type=jnp.float32) + ba2_ref[...]
        a3 = t.reshape(R, K, C)
        mx = jnp.max(a3, axis=1, keepdims=True)
        e = jnp.exp(a3 - mx)
        p = e / jnp.sum(e, axis=1, keepdims=True)
        vpe = (vn + pe).reshape(R, K, C)
        agg = jnp.sum(p * vpe, axis=1)
        attn = jnp.dot(agg, wo_ref[...],
                       preferred_element_type=jnp.float32) + bo_ref[...]
        x1 = xm_ref[0] + attn
        h = jnp.dot(x1, wf1_ref[...],
                    preferred_element_type=jnp.float32) + bf1_ref[...]
        h = _ln(h, g1_ref[...], be1_ref[...])
        h = jax.nn.gelu(h)
        h = jnp.dot(h, wf2_ref[...],
                    preferred_element_type=jnp.float32) + bf2_ref[...]
        h = _ln(h, g2_ref[...], be2_ref[...])
        out = x1 + h
        if nxt is not None:
            _mod_qkv_refs(out, e_refs, o_refs)
        elif proj is not None:
            o_refs[0][0] = jnp.dot(out, e_refs[0][...],
                                   preferred_element_type=jnp.float32) \
                + e_refs[1][...]
        else:
            o_refs[0][0] = out

    res = pl.pallas_call(
        body,
        grid=(B_, N_ // R),
        in_specs=[
            pl.BlockSpec((1, R, C), lambda b, i: (b, i, 0)),
            pl.BlockSpec((1, R, C), lambda b, i: (b, i, 0)),
            pl.BlockSpec((1, RK, 2 * C), lambda b, i: (b, i, 0)),
            pl.BlockSpec((1, RK, 16), lambda b, i: (b, i, 0)),
        ] + [_full(w.shape) for w in wts] + e_specs,
        out_specs=_row_specs(oshapes, R),
        out_shape=oshapes,
    )(xm, q, kvn, pd, *wts, *extra)
    return res[0] if len(res) == 1 else tuple(res)


def _slot(x, pos16, sp, wd, bd, nxt):
    B_, N_, C = x.shape
    S = sp['slots'].shape[0]
    Cd = wd.shape[1]

    wlist = [sp['slots'],
             sp['wq']['w'], sp['wq']['b'][None, :],
             sp['wk']['w'], sp['wk']['b'][None, :],
             sp['wv']['w'], sp['wv']['b'][None, :],
             sp['mlp1']['w'], sp['mlp1']['b'][None, :],
             sp['mlp2']['w'], sp['mlp2']['b'][None, :],
             wd, bd]

    def body(*refs):
        x_ref, pos_ref = refs[:2]
        (slots_ref, wq_ref, bq_ref, wk_ref, bk_ref, wv_ref, bv_ref,
         m1_ref, bm1_ref, m2_ref, bm2_ref, wd_ref, bd_ref) = refs[2:15]
        e_refs = refs[15:21]
        xm_ref, q2_ref, kv2_ref, posn_ref = refs[21:]
        x_ = x_ref[0]
        qs = jnp.dot(slots_ref[...], wq_ref[...],
                     preferred_element_type=jnp.float32) + bq_ref[...]
        kk = jnp.dot(x_, wk_ref[...],
                     preferred_element_type=jnp.float32) + bk_ref[...]
        vv = jnp.dot(x_, wv_ref[...],
                     preferred_element_type=jnp.float32) + bv_ref[...]
        logits = lax.dot_general(qs, kk, (((1,), (1,)), ((), ())),
                                 preferred_element_type=jnp.float32) / jnp.sqrt(
                                     jnp.float32(C))
        mx = jnp.max(logits, axis=0, keepdims=True)
        e = jnp.exp(logits - mx)
        attn = e / jnp.sum(e, axis=0, keepdims=True)
        w = attn / (jnp.sum(attn, axis=1, keepdims=True) + 1e-8)
        upd = jnp.dot(w, vv, preferred_element_type=jnp.float32)
        t = jnp.dot(jax.nn.relu(
            jnp.dot(upd, m1_ref[...], preferred_element_type=jnp.float32)
            + bm1_ref[...]), m2_ref[...],
            preferred_element_type=jnp.float32) + bm2_ref[...]
        xn = upd + t
        xd = jnp.dot(xn, wd_ref[...],
                     preferred_element_type=jnp.float32) + bd_ref[...]
        _mod_qkv_refs(xd, e_refs, (xm_ref, q2_ref, kv2_ref))
        posn_ref[0] = jnp.dot(w, pos_ref[0],
                              preferred_element_type=jnp.float32)

    oshapes = _qkv_shapes(B_, S, Cd) + [
        jax.ShapeDtypeStruct((B_, S, 16), jnp.float32)]
    return pl.pallas_call(
        body,
        grid=(B_, 1),
        in_specs=[
            pl.BlockSpec((1, N_, C), lambda b, i: (b, 0, 0)),
            pl.BlockSpec((1, N_, 16), lambda b, i: (b, 0, 0)),
        ] + [_full(w.shape) for w in wlist] + _nxt_specs(nxt, Cd),
        out_specs=_row_specs(oshapes, S),
        out_shape=oshapes,
    )(x, pos16, *wlist, *_nxt_arrays(nxt))


def _interp(aug_a, aug_b, x_src, x_skip, wu, bu, nxt, R):
    B_, N_, Cs = x_skip.shape
    S = x_src.shape[1]

    def body(*refs):
        a_ref, b_ref, xsrc_ref, xs_ref, wu_ref, bu_ref = refs[:6]
        e_refs = refs[6:12]
        o_refs = refs[12:]
        d = lax.dot_general(a_ref[0], b_ref[0], (((1,), (1,)), ((), ())),
                            preferred_element_type=jnp.float32)
        col = lax.broadcasted_iota(jnp.int32, (R, S), 1)
        dbits = lax.bitcast_convert_type(d, jnp.int32)
        packed = (dbits & jnp.int32(~0x7FF)) | col
        dead = jnp.int32(0x7FFFFFFF)
        keys = []
        for _ in range(K):
            m = jnp.min(packed, axis=1, keepdims=True)
            keys.append(m)
            packed = jnp.where(packed == m, dead, packed)
        kcat = jnp.concatenate(keys, axis=1)
        d2 = lax.bitcast_convert_type(kcat & jnp.int32(~0x7FF), jnp.float32)
        dist = jnp.sqrt(jnp.maximum(d2, 0.0) + 1e-12)
        w = 1.0 / (dist + 1e-8)
        w = w / jnp.sum(w, axis=1, keepdims=True)
        wm = jnp.zeros((R, S), jnp.float32)
        for r in range(K):
            am = kcat[:, r:r + 1] & jnp.int32(0x7FF)
            wm = jnp.where(col == am, w[:, r:r + 1], wm)
        itp = jnp.dot(wm, xsrc_ref[0],
                      preferred_element_type=jnp.float32)
        out = jnp.dot(itp, wu_ref[...],
                      preferred_element_type=jnp.float32) \
            + bu_ref[...] + xs_ref[0]
        _mod_qkv_refs(out, e_refs, o_refs)

    oshapes = _qkv_shapes(B_, N_, Cs)
    return pl.pallas_call(
        body,
        grid=(B_, N_ // R),
        in_specs=[
            pl.BlockSpec((1, R, 8), lambda b, i: (b, i, 0)),
            pl.BlockSpec((1, S, 8), lambda b, i: (b, 0, 0)),
            pl.BlockSpec((1, S, x_src.shape[2]), lambda b, i: (b, 0, 0)),
            pl.BlockSpec((1, R, Cs), lambda b, i: (b, i, 0)),
            _full(wu.shape), _full(bu.shape),
        ] + _nxt_specs(nxt, Cs),
        out_specs=_row_specs(oshapes, R),
        out_shape=oshapes,
    )(aug_a, aug_b, x_src, x_skip, wu, bu, *_nxt_arrays(nxt))


def _prep_pos(p):
    n2 = jnp.sum(p * p, axis=-1, keepdims=True)
    one = jnp.ones_like(n2)
    z = jnp.zeros_like(n2)
    z3 = jnp.concatenate([z, z, z], -1)
    aug_a = jnp.concatenate([p, n2, one, z3], -1)
    aug_b = jnp.concatenate([-2.0 * p, one, n2, z3], -1)
    pos16 = jnp.concatenate([p, z, z3, z3, z3, z3], -1)
    return aug_a, aug_b, pos16


def _adapt_np(cond, p):
    h = cond @ p['lin1']['w'] + p['lin1']['b']
    m = jnp.mean(h, axis=-1, keepdims=True)
    v = jnp.var(h, axis=-1, keepdims=True)
    h = (h - m) / jnp.sqrt(v + 1e-5) * p['ln']['g'] + p['ln']['b']
    h = h @ p['lin2']['w'] + p['lin2']['b']
    sh, sc = jnp.split(h, 2, axis=1)
    return sh[:, None, :], sc[:, None, :]


def _pad_rows(w, rows):
    return jnp.concatenate(
        [w, jnp.zeros((rows - w.shape[0], w.shape[1]), w.dtype)], axis=0)


def _pad_cols(w, cols):
    return jnp.concatenate(
        [w, jnp.zeros((w.shape[0], cols - w.shape[1]), w.dtype)], axis=1)


def _layer_wts(p):
    return [
        _pad_rows(p['pos1']['w'], 16), p['pos1']['b'][None, :],
        p['pos2']['w'], p['pos2']['b'][None, :],
        p['attn1']['w'], p['attn1']['b'][None, :],
        p['attn2']['w'], p['attn2']['b'][None, :],
        p['fc_out']['w'], p['fc_out']['b'][None, :],
        p['ffn1']['w'], p['ffn1']['b'][None, :],
        p['ffn2']['w'], p['ffn2']['b'][None, :],
        p['ffn_ln1']['g'][None, :], p['ffn_ln1']['b'][None, :],
        p['ffn_ln2']['g'][None, :], p['ffn_ln2']['b'][None, :],
    ]


def _nxt_of(sh_sc, lp):
    sh, sc = sh_sc
    return (sh, sc, lp['fc_q'], lp['fc_kv'])


def kernel(pos, cond, params):
    outs = [_forward_one(pos[b:b + 1], cond[b:b + 1], params)
            for b in range(pos.shape[0])]
    return jnp.concatenate(outs, axis=0)


def _forward_one(pos, cond, params):
    B_, N_, _ = pos.shape
    C0, C1 = 64, 128
    S = 512
    R0, R1 = 512, 256

    aug_a0, aug_b0, pos16_0 = _prep_pos(pos)

    sh_sc_down = [_adapt_np(cond, p) for p in params['adapt_down']]
    sh_sc_up = [_adapt_np(cond, p) for p in params['adapt_up']]

    td0, td1 = params['tf_down'][0], params['tf_down'][1]
    tu0 = params['tf_up'][0]

    win = _pad_rows(params['proj_in']['w'], 8)
    xm, q, kv = _projin_qkv(aug_a0, win, params['proj_in']['b'][None, :],
                            _nxt_of(sh_sc_down[0], td0[0]), R0)

    idx0, pd0 = _knn(aug_a0, aug_b0, pos16_0, pos16_0, 512)
    idx0_flat = idx0.reshape(-1)
    pd0 = pd0.reshape(B_, N_ * K, 16)

    kvn = _gather_kv(kv, idx0_flat)
    xm, q, kv = _attn_ffn(xm, q, kvn, pd0, _layer_wts(td0[0]), R0,
                          nxt=_nxt_of(sh_sc_down[1], td0[1]))
    kvn = _gather_kv(kv, idx0_flat)
    x0_skip = _attn_ffn(xm, q, kvn, pd0, _layer_wts(td0[1]), R0)

    xm, q, kv, pos1_16 = _slot(
        x0_skip, pos16_0, params['down'][0],
        params['proj_down'][0]['w'], params['proj_down'][0]['b'][None, :],
        _nxt_of(sh_sc_down[2], td1[0]))
    pos1 = pos1_16[..., :3]
    aug_a1, aug_b1, pos16_1 = _prep_pos(pos1)

    idx1, pd1 = _knn(aug_a1, aug_b1, pos16_1, pos16_1, 512)
    idx1_flat = idx1.reshape(-1)
    pd1 = pd1.reshape(B_, S * K, 16)

    kvn = _gather_kv(kv, idx1_flat)
    xm, q, kv = _attn_ffn(xm, q, kvn, pd1, _layer_wts(td1[0]), R1,
                          nxt=_nxt_of(sh_sc_down[3], td1[1]))
    kvn = _gather_kv(kv, idx1_flat)
    x1_out = _attn_ffn(xm, q, kvn, pd1, _layer_wts(td1[1]), R1)

    xm, q, kv = _interp(aug_a0, aug_b1, x1_out, x0_skip,
                        params['proj_up'][0]['w'],
                        params['proj_up'][0]['b'][None, :],
                        _nxt_of(sh_sc_up[-1], tu0[0]), R0)

    kvn = _gather_kv(kv, idx0_flat)
    xm, q, kv = _attn_ffn(xm, q, kvn, pd0, _layer_wts(tu0[0]), R0,
                          nxt=_nxt_of(sh_sc_up[0], tu0[1]))
    kvn = _gather_kv(kv, idx0_flat)

    wout = _pad_cols(params['proj_out']['w'], 128)
    bout = _pad_cols(params['proj_out']['b'][None, :], 128)
    out = _attn_ffn(xm, q, kvn, pd0, _layer_wts(tu0[1]), R0,
                    proj=(wout, bout))
    return out[..., :1]
